# baseline (speedup 1.0000x reference)
import functools

import jax
import jax.numpy as jnp
from jax import lax
from jax.experimental import pallas as pl
from jax.experimental.pallas import tpu as pltpu
from jax.experimental.pallas import tpu_sc as plsc

_MAXLEN = 200
_EMBED = 64
_BATCH = 4096
_NC = 2
_NS = 16
_NW = _NC * _NS
_SEQ_PER_W = _BATCH // _NW
_CHUNK_A = 128
_CHUNK_B = _MAXLEN - _CHUNK_A
_NBUF = 8


def _make_sc_kernel():
    mesh = plsc.VectorSubcoreMesh(core_axis_name="c", subcore_axis_name="s")

    @functools.partial(
        pl.kernel,
        mesh=mesh,
        compiler_params=pltpu.CompilerParams(use_tc_tiling_on_sc=False),
        out_type=jax.ShapeDtypeStruct((_BATCH, _MAXLEN, _EMBED), jnp.float32),
        scratch_types=[
            pltpu.VMEM_SHARED((_MAXLEN, _EMBED), jnp.float32),
            pltpu.VMEM((_SEQ_PER_W, _MAXLEN), jnp.int32),
            pltpu.VMEM((_NBUF, _MAXLEN, _EMBED), jnp.float32),
        ]
        + [pltpu.SemaphoreType.DMA] * (3 * _NBUF),
    )
    def k(x_hbm, tok_hbm, pos_hbm, out_hbm, pos_sh, idx_v, rows_v, *sems):
        isems, gsems, osems = sems[:_NBUF], sems[_NBUF:2 * _NBUF], sems[2 * _NBUF:]
        sid = lax.axis_index("s")
        wid = sid * _NC + lax.axis_index("c")
        seq0 = wid * _SEQ_PER_W

        @pl.when(sid == 0)
        def _():
            pltpu.sync_copy(pos_hbm, pos_sh)
        plsc.subcore_barrier()
        pltpu.sync_copy(x_hbm.at[pl.ds(seq0, _SEQ_PER_W)], idx_v)

        def fire_init(b):
            pltpu.async_copy(pos_sh, rows_v.at[b], isems[b])

        def wait_init(b):
            pltpu.make_async_copy(pos_sh, rows_v.at[b], isems[b]).wait()

        def fire_gather(i, b):
            pltpu.async_copy(tok_hbm.at[idx_v.at[i, pl.ds(0, _CHUNK_A)]],
                             rows_v.at[b, pl.ds(0, _CHUNK_A)], gsems[b],
                             add=True)
            pltpu.async_copy(tok_hbm.at[idx_v.at[i, pl.ds(_CHUNK_A, _CHUNK_B)]],
                             rows_v.at[b, pl.ds(_CHUNK_A, _CHUNK_B)], gsems[b],
                             add=True)

        def wait_gather(b):
            pltpu.make_async_copy(tok_hbm.at[idx_v.at[0, pl.ds(0, _CHUNK_A)]],
                                  rows_v.at[b, pl.ds(0, _CHUNK_A)],
                                  gsems[b]).wait()
            pltpu.make_async_copy(tok_hbm.at[idx_v.at[0, pl.ds(0, _CHUNK_B)]],
                                  rows_v.at[b, pl.ds(_CHUNK_A, _CHUNK_B)],
                                  gsems[b]).wait()

        def fire_store(i, b):
            pltpu.async_copy(rows_v.at[b], out_hbm.at[seq0 + i], osems[b])

        def wait_store(i, b):
            pltpu.make_async_copy(rows_v.at[b], out_hbm.at[seq0 + i],
                                  osems[b]).wait()

        for b in range(_NBUF):
            fire_init(b)
        for b in range(_NBUF):
            wait_init(b)
            fire_gather(b, b)

        def outer(g, carry):
            i0 = g * _NBUF
            for b in range(_NBUF):
                wait_gather(b)
                fire_store(i0 + b, b)
            for b in range(_NBUF):
                wait_store(i0 + b, b)
                fire_init(b)
            for b in range(_NBUF):
                wait_init(b)
                fire_gather(i0 + _NBUF + b, b)
            return carry

        lax.fori_loop(0, _SEQ_PER_W // _NBUF - 1, outer, 0)

        i0 = _SEQ_PER_W - _NBUF
        for b in range(_NBUF):
            wait_gather(b)
            fire_store(i0 + b, b)
        for b in range(_NBUF):
            wait_store(i0 + b, b)

    return k


def kernel(x, token_table, pos_table):
    return _make_sc_kernel()(x.astype(jnp.int32), token_table, pos_table)


# speedup vs baseline: 1.0878x; 1.0878x over previous
import functools

import jax
import jax.numpy as jnp
from jax import lax
from jax.experimental import pallas as pl
from jax.experimental.pallas import tpu as pltpu
from jax.experimental.pallas import tpu_sc as plsc

_MAXLEN = 200
_EMBED = 64
_BATCH = 4096
_NC = 2
_NS = 16
_NW = _NC * _NS
_SEQ_PER_W = _BATCH // _NW
_CHUNK_A = 128
_CHUNK_B = _MAXLEN - _CHUNK_A
_NBUF = 2
_VL = 16


def _make_sc_kernel():
    mesh = plsc.VectorSubcoreMesh(core_axis_name="c", subcore_axis_name="s")

    @functools.partial(
        pl.kernel,
        mesh=mesh,
        compiler_params=pltpu.CompilerParams(use_tc_tiling_on_sc=True),
        out_type=jax.ShapeDtypeStruct((_BATCH, _MAXLEN, _EMBED), jnp.float32),
        scratch_types=[
            pltpu.VMEM_SHARED((_MAXLEN, 2 * _EMBED), jnp.float32),
            pltpu.VMEM((_SEQ_PER_W * _MAXLEN,), jnp.int32),
            pltpu.VMEM((_NBUF, _MAXLEN, 2 * _EMBED), jnp.float32),
            pltpu.VMEM((_NBUF, _MAXLEN, _EMBED), jnp.float32),
        ]
        + [pltpu.SemaphoreType.DMA] * (2 * _NBUF),
    )
    def k(x_hbm, tok_hbm, pos_hbm, out_hbm, pos_sh, idx_v, gbuf, sbuf, *sems):
        gsems, osems = sems[:_NBUF], sems[_NBUF:]
        sid = lax.axis_index("s")
        wid = sid * _NC + lax.axis_index("c")
        seq0 = wid * _SEQ_PER_W

        @pl.when(sid == 0)
        def _():
            pltpu.sync_copy(pos_hbm, pos_sh)
        plsc.subcore_barrier()
        pltpu.sync_copy(x_hbm.at[pl.ds(seq0 * _MAXLEN, _SEQ_PER_W * _MAXLEN)],
                        idx_v)

        def fire_init_gather(i, b):
            pltpu.sync_copy(pos_sh, gbuf.at[b])
            pltpu.async_copy(tok_hbm.at[idx_v.at[pl.ds(i * _MAXLEN, _CHUNK_A)]],
                             gbuf.at[b, pl.ds(0, _CHUNK_A)], gsems[b],
                             add=True)
            pltpu.async_copy(
                tok_hbm.at[idx_v.at[pl.ds(i * _MAXLEN + _CHUNK_A, _CHUNK_B)]],
                gbuf.at[b, pl.ds(_CHUNK_A, _CHUNK_B)], gsems[b],
                add=True)

        def wait_gather(b):
            pltpu.make_async_copy(tok_hbm.at[idx_v.at[pl.ds(0, _CHUNK_A)]],
                                  gbuf.at[b, pl.ds(0, _CHUNK_A)],
                                  gsems[b]).wait()
            pltpu.make_async_copy(tok_hbm.at[idx_v.at[pl.ds(0, _CHUNK_B)]],
                                  gbuf.at[b, pl.ds(_CHUNK_A, _CHUNK_B)],
                                  gsems[b]).wait()

        def depad(b):
            def row(s, carry):
                for j in range(_EMBED // _VL):
                    sbuf[b, s, pl.ds(j * _VL, _VL)] = (
                        gbuf[b, s, pl.ds(j * _VL, _VL)])
                return carry
            lax.fori_loop(0, _MAXLEN, row, 0)

        def fire_store(i, b):
            pltpu.async_copy(sbuf.at[b], out_hbm.at[seq0 + i], osems[b])

        def wait_store(i, b):
            pltpu.make_async_copy(sbuf.at[b], out_hbm.at[seq0 + i],
                                  osems[b]).wait()

        for b in range(_NBUF):
            fire_init_gather(b, b)

        def outer(g, carry):
            i0 = g * _NBUF
            for b in range(_NBUF):
                i = i0 + b
                wait_gather(b)
                depad(b)
                @pl.when(i + _NBUF < _SEQ_PER_W)
                def _():
                    fire_init_gather(i + _NBUF, b)
                @pl.when(g > 0)
                def _():
                    wait_store(i - _NBUF, b)
                fire_store(i, b)
            return carry

        lax.fori_loop(0, _SEQ_PER_W // _NBUF, outer, 0)

        for b in range(_NBUF):
            wait_store(_SEQ_PER_W - _NBUF + b, b)

    return k


def kernel(x, token_table, pos_table):
    x_flat = x.astype(jnp.int32).reshape(-1)
    tok_p = jnp.pad(token_table, ((0, 0), (0, _EMBED)))
    pos_p = jnp.pad(pos_table, ((0, 0), (0, _EMBED)))
    return _make_sc_kernel()(x_flat, tok_p, pos_p)


# speedup vs baseline: 1.1006x; 1.0118x over previous
import functools

import jax
import jax.numpy as jnp
from jax import lax
from jax.experimental import pallas as pl
from jax.experimental.pallas import tpu as pltpu
from jax.experimental.pallas import tpu_sc as plsc

_MAXLEN = 200
_EMBED = 64
_BATCH = 4096
_NC = 2
_NS = 16
_NW = _NC * _NS
_SEQ_PER_W = _BATCH // _NW
_HA = 96
_HB = _MAXLEN - _HA
_NCHUNK = 2 * _SEQ_PER_W
_NGB = 4
_NSB = 2
_VL = 16
_RU = 8


def _chunk_rows(b):
    return _HA if b % 2 == 0 else _HB


def _make_sc_kernel():
    mesh = plsc.VectorSubcoreMesh(core_axis_name="c", subcore_axis_name="s")

    @functools.partial(
        pl.kernel,
        mesh=mesh,
        compiler_params=pltpu.CompilerParams(use_tc_tiling_on_sc=True),
        out_type=jax.ShapeDtypeStruct((_BATCH, _MAXLEN, _EMBED), jnp.float32),
        scratch_types=[
            pltpu.VMEM_SHARED((_MAXLEN, 2 * _EMBED), jnp.float32),
            pltpu.VMEM((_SEQ_PER_W * _MAXLEN,), jnp.int32),
            pltpu.VMEM((_NGB, _HB, 2 * _EMBED), jnp.float32),
            pltpu.VMEM((_NSB, _HB, _EMBED), jnp.float32),
        ]
        + [pltpu.SemaphoreType.DMA] * (2 * _NGB + _NSB),
    )
    def k(x_hbm, tok_hbm, pos_hbm, out_hbm, pos_sh, idx_v, gbuf, sbuf, *sems):
        isems = sems[:_NGB]
        gsems = sems[_NGB:2 * _NGB]
        osems = sems[2 * _NGB:]
        sid = lax.axis_index("s")
        wid = sid * _NC + lax.axis_index("c")
        seq0 = wid * _SEQ_PER_W

        @pl.when(sid == 0)
        def _():
            pltpu.sync_copy(pos_hbm, pos_sh)
        plsc.subcore_barrier()
        pltpu.sync_copy(x_hbm.at[pl.ds(seq0 * _MAXLEN, _SEQ_PER_W * _MAXLEN)],
                        idx_v)


        def fire_init(b):
            rows = _chunk_rows(b)
            pltpu.async_copy(pos_sh.at[pl.ds((b % 2) * _HA, rows)],
                             gbuf.at[b, pl.ds(0, rows)], isems[b])

        def wait_init(b):
            rows = _chunk_rows(b)
            pltpu.make_async_copy(pos_sh.at[pl.ds((b % 2) * _HA, rows)],
                                  gbuf.at[b, pl.ds(0, rows)], isems[b]).wait()

        def fire_gather(k, b):
            rows = _chunk_rows(b)
            i0 = (k // 2) * _MAXLEN + (b % 2) * _HA
            pltpu.async_copy(tok_hbm.at[idx_v.at[pl.ds(i0, rows)]],
                             gbuf.at[b, pl.ds(0, rows)], gsems[b],
                             add=True)

        def wait_gather(b):
            rows = _chunk_rows(b)
            pltpu.make_async_copy(tok_hbm.at[idx_v.at[pl.ds(0, rows)]],
                                  gbuf.at[b, pl.ds(0, rows)],
                                  gsems[b]).wait()

        def depad(b, s):
            rows = _chunk_rows(b)

            def body(t, carry):
                for r in range(_RU):
                    for j in range(_EMBED // _VL):
                        sbuf[s, t * _RU + r, pl.ds(j * _VL, _VL)] = (
                            gbuf[b, t * _RU + r, pl.ds(j * _VL, _VL)])
                return carry

            lax.fori_loop(0, rows // _RU, body, 0)

        def fire_store(k, s):
            rows = _chunk_rows(s)
            pltpu.async_copy(
                sbuf.at[s, pl.ds(0, rows)],
                out_hbm.at[seq0 + k // 2, pl.ds((s % 2) * _HA, rows)],
                osems[s])

        def wait_store(k, s):
            rows = _chunk_rows(s)
            pltpu.make_async_copy(
                sbuf.at[s, pl.ds(0, rows)],
                out_hbm.at[seq0 + k // 2, pl.ds((s % 2) * _HA, rows)],
                osems[s]).wait()

        for b in range(_NGB):
            fire_init(b)
        for b in range(2):
            wait_init(b)
            fire_gather(b, b)

        def outer(g, carry):
            k0 = g * _NGB
            for b in range(_NGB):
                k = k0 + b
                s = b % _NSB
                c = (b + 2) % _NGB
                @pl.when(k + 2 < _NCHUNK)
                def _():
                    wait_init(c)
                    fire_gather(k + 2, c)
                wait_gather(b)
                @pl.when(k >= _NSB)
                def _():
                    wait_store(k - _NSB, s)
                depad(b, s)
                @pl.when(k + _NGB < _NCHUNK)
                def _():
                    fire_init(b)
                fire_store(k, s)
            return carry

        lax.fori_loop(0, _NCHUNK // _NGB, outer, 0)

        for s in range(_NSB):
            wait_store(_NCHUNK - _NSB + s, s)

    return k


def kernel(x, token_table, pos_table):
    x_flat = x.astype(jnp.int32).reshape(-1)
    tok_p = jnp.pad(token_table, ((0, 0), (0, _EMBED)))
    pos_p = jnp.pad(pos_table, ((0, 0), (0, _EMBED)))
    return _make_sc_kernel()(x_flat, tok_p, pos_p)
